# baseline (speedup 1.0000x reference)
import functools

import jax
import jax.numpy as jnp
from jax import lax
from jax.experimental import pallas as pl
from jax.experimental.pallas import tpu as pltpu
from jax.experimental.pallas import tpu_sc as plsc

B = 4096
EMB = 64
H = 2 * EMB

_NC = 2
_NS = 16
_NW = _NC * _NS
_BPW = B // _NW


def _sc_gather(track_table, artist_table, w1_genre, track_id, artist_id, genres):
  mesh = plsc.VectorSubcoreMesh(core_axis_name="c", subcore_axis_name="s")

  @functools.partial(
      pl.kernel,
      mesh=mesh,
      compiler_params=pltpu.CompilerParams(use_tc_tiling_on_sc=False),
      out_type=(
          jax.ShapeDtypeStruct((B, EMB), jnp.float32),
          jax.ShapeDtypeStruct((B, EMB), jnp.float32),
          jax.ShapeDtypeStruct((B, H), jnp.float32),
      ),
      scratch_types=[
          pltpu.VMEM((_BPW,), jnp.int32),
          pltpu.VMEM((_BPW,), jnp.int32),
          pltpu.VMEM((_BPW,), jnp.int32),
          pltpu.VMEM((_BPW, EMB), jnp.float32),
          pltpu.VMEM((_BPW, EMB), jnp.float32),
          pltpu.VMEM((_BPW, H), jnp.float32),
          pltpu.SemaphoreType.DMA,
      ],
  )
  def k(tt, at, wg, tid, aid, gid, t_out, a_out, g_out,
        tix, aix, gix, trows, arows, grows, sem):
    wid = lax.axis_index("s") * _NC + lax.axis_index("c")
    base = wid * _BPW
    pltpu.sync_copy(tid.at[pl.ds(base, _BPW)], tix)
    pltpu.sync_copy(aid.at[pl.ds(base, _BPW)], aix)
    pltpu.sync_copy(gid.at[pl.ds(base, _BPW)], gix)
    c1 = pltpu.async_copy(tt.at[tix], trows, sem)
    c2 = pltpu.async_copy(at.at[aix], arows, sem)
    c3 = pltpu.async_copy(wg.at[gix], grows, sem)
    c1.wait()
    c2.wait()
    c3.wait()
    pltpu.sync_copy(trows, t_out.at[pl.ds(base, _BPW)])
    pltpu.sync_copy(arows, a_out.at[pl.ds(base, _BPW)])
    pltpu.sync_copy(grows, g_out.at[pl.ds(base, _BPW)])

  return k(track_table, artist_table, w1_genre, track_id, artist_id, genres)


_BLK = 512


def _tc_body(t_ref, a_ref, g_ref, au_ref, mean_ref, var_ref,
             w1t_ref, w1a_ref, w1f_ref, b1_ref, w2_ref, b2_ref, o_ref):
  audio = (au_ref[...] - mean_ref[...]) * lax.rsqrt(var_ref[...])
  h = g_ref[...] + b1_ref[...]
  h += jnp.dot(t_ref[...], w1t_ref[...], preferred_element_type=jnp.float32)
  h += jnp.dot(a_ref[...], w1a_ref[...], preferred_element_type=jnp.float32)
  h += jnp.dot(audio, w1f_ref[...], preferred_element_type=jnp.float32)
  h = jnp.maximum(h, 0.0)
  o_ref[...] = jnp.dot(h, w2_ref[...], preferred_element_type=jnp.float32) + b2_ref[...]


def _tc_dense(trows, arows, grows, audio, norm_mean, norm_var,
              w1_t, w1_a, w1_f, b1, W2, b2):
  n_blk = B // _BLK
  full = lambda shape: pl.BlockSpec(shape, lambda i: (0, 0))
  return pl.pallas_call(
      _tc_body,
      grid=(n_blk,),
      in_specs=[
          pl.BlockSpec((_BLK, EMB), lambda i: (i, 0)),
          pl.BlockSpec((_BLK, EMB), lambda i: (i, 0)),
          pl.BlockSpec((_BLK, H), lambda i: (i, 0)),
          pl.BlockSpec((_BLK, 8), lambda i: (i, 0)),
          full((1, 8)),
          full((1, 8)),
          full((EMB, H)),
          full((EMB, H)),
          full((8, H)),
          full((1, H)),
          full((H, EMB)),
          full((1, EMB)),
      ],
      out_specs=pl.BlockSpec((_BLK, EMB), lambda i: (i, 0)),
      out_shape=jax.ShapeDtypeStruct((B, EMB), jnp.float32),
  )(trows, arows, grows, audio, norm_mean, norm_var,
    w1_t, w1_a, w1_f, b1, W2, b2)


def kernel(track_id, artist_id, genres, danceability, energy, instrumentalness,
           acousticness, valence, speechiness, loudness, liveness,
           norm_mean, norm_var, track_table, artist_table, W1, b1, W2, b2):
  tid = track_id.astype(jnp.int32)
  aid = artist_id.astype(jnp.int32)
  gid = genres.astype(jnp.int32)
  w1_genre = W1[2 * EMB:2 * EMB + 101, :]
  trows, arows, grows = _sc_gather(track_table, artist_table, w1_genre,
                                   tid, aid, gid)
  audio = jnp.stack([danceability, energy, instrumentalness, acousticness,
                     valence, speechiness, loudness, liveness], axis=1)
  return _tc_dense(trows, arows, grows, audio,
                   norm_mean.reshape(1, 8), norm_var.reshape(1, 8),
                   W1[:EMB, :], W1[EMB:2 * EMB, :], W1[2 * EMB + 101:, :],
                   b1.reshape(1, H), W2, b2.reshape(1, EMB))
